# baseline (speedup 1.0000x reference)
import functools

import jax
import jax.numpy as jnp
from jax import lax
from jax.experimental import pallas as pl
from jax.experimental.pallas import tpu as pltpu
from jax.experimental.pallas import tpu_sc as plsc

B = 128
G = 8192
NUM_MASKED = 4915
L = 16
CHUNKS = G // L
NBITS = 10
NB = 1 << NBITS
HCHUNKS = NB // L
N_WORKERS = 32
ROWS_PER_TILE = B // N_WORKERS

_mesh = plsc.VectorSubcoreMesh(core_axis_name="c", subcore_axis_name="s")


@functools.partial(
    pl.kernel,
    out_type=jax.ShapeDtypeStruct((B, G), jnp.int32),
    mesh=_mesh,
    scratch_types=[
        pltpu.VMEM((G,), jnp.float32),
        pltpu.VMEM((G,), jnp.int32),
        pltpu.VMEM((G,), jnp.int32),
        pltpu.VMEM((NB,), jnp.int32),
        pltpu.VMEM((L,), jnp.int32),
    ],
    compiler_params=pltpu.CompilerParams(needs_layout_passes=False),
)
def _argsort_rows(noise_hbm, out_hbm, noise_v, idx_a, idx_b, hist, sd_v):
    core = lax.axis_index("c")
    sub = lax.axis_index("s")
    wid = sub * 2 + core
    iota = lax.iota(jnp.int32, L)

    def chunk_stats(d, idxv):
        key2 = (d << 4) | iota
        sk, sv = plsc.sort_key_val(key2, idxv)
        sd = sk >> 4
        sd_v[...] = sd
        prev = plsc.load_gather(sd_v, [jnp.maximum(iota - 1, 0)])
        nxt = plsc.load_gather(sd_v, [jnp.minimum(iota + 1, L - 1)])
        is_new = (iota == 0) | (prev != sd)
        is_end = (iota == L - 1) | (nxt != sd)
        start = plsc.cummax(jnp.where(is_new, iota, 0))
        rank = iota - start
        return sd, sv, rank, is_end

    def do_row(r, _):
        row = wid * ROWS_PER_TILE + r
        pltpu.sync_copy(noise_hbm.at[row], noise_v)

        for p in range(3):
            shift = NBITS * p
            src = idx_a if p == 1 else idx_b
            dst = idx_b if p == 1 else idx_a

            def load_chunk(c):
                if p == 0:
                    idxv = c * L + iota
                else:
                    idxv = src[pl.ds(c * L, L)]
                kf = plsc.load_gather(noise_v, [idxv])
                kv = plsc.bitcast(kf, jnp.int32)
                d = lax.shift_right_logical(kv, shift) & (NB - 1)
                return d, idxv

            def clr(i, carry):
                hist[pl.ds(i * L, L)] = jnp.zeros((L,), jnp.int32)
                return carry

            lax.fori_loop(0, HCHUNKS, clr, jnp.int32(0))

            def histo(c, carry):
                d, idxv = load_chunk(c)
                sd, _, rank, is_end = chunk_stats(d, idxv)
                plsc.addupdate_scatter(hist, [sd], rank + 1, mask=is_end)
                return carry

            lax.fori_loop(0, CHUNKS, histo, jnp.int32(0))

            def scan(i, carry):
                v = hist[pl.ds(i * L, L)]
                incl = plsc.cumsum(v)
                hist[pl.ds(i * L, L)] = incl - v + carry
                return carry + jnp.max(incl)

            lax.fori_loop(0, HCHUNKS, scan, jnp.int32(0))

            def scat(c, carry):
                d, idxv = load_chunk(c)
                sd, sv, rank, is_end = chunk_stats(d, idxv)
                starts = plsc.load_gather(hist, [sd])
                plsc.store_scatter(dst, [starts + rank], sv)
                plsc.addupdate_scatter(hist, [sd], rank + 1, mask=is_end)
                return carry

            lax.fori_loop(0, CHUNKS, scat, jnp.int32(0))

        pltpu.sync_copy(idx_a, out_hbm.at[row])
        return jnp.int32(0)

    lax.fori_loop(0, ROWS_PER_TILE, do_row, jnp.int32(0))


def kernel(x, noise):
    del x
    perm = _argsort_rows(noise)
    return perm[:, :NUM_MASKED], perm[:, NUM_MASKED:]


# speedup vs baseline: 1.4149x; 1.4149x over previous
import functools

import jax
import jax.numpy as jnp
from jax import lax
from jax.experimental import pallas as pl
from jax.experimental.pallas import tpu as pltpu
from jax.experimental.pallas import tpu_sc as plsc

B = 128
G = 8192
NUM_MASKED = 4915
L = 16
CHUNKS = G // L
NBITS = 10
NB = 1 << NBITS
HCHUNKS = NB // L
N_WORKERS = 32
ROWS_PER_TILE = B // N_WORKERS

_mesh = plsc.VectorSubcoreMesh(core_axis_name="c", subcore_axis_name="s")


@functools.partial(
    pl.kernel,
    out_type=jax.ShapeDtypeStruct((B, G), jnp.int32),
    mesh=_mesh,
    scratch_types=[
        pltpu.VMEM((G,), jnp.float32),
        pltpu.VMEM((G,), jnp.int32),
        pltpu.VMEM((G,), jnp.int32),
        pltpu.VMEM((NB,), jnp.int32),
    ],
    compiler_params=pltpu.CompilerParams(needs_layout_passes=False),
)
def _argsort_rows(noise_hbm, out_hbm, noise_v, idx_a, idx_b, hist):
    core = lax.axis_index("c")
    sub = lax.axis_index("s")
    wid = sub * 2 + core
    iota = lax.iota(jnp.int32, L)

    cnt0, _ = plsc.scan_count(jnp.zeros((L,), jnp.int32))
    c0 = jnp.min(cnt0)

    def chunk_stats(d, idxv):
        cnt, last = plsc.scan_count(d)
        return d, idxv, cnt - c0, last

    def do_row(r, _):
        row = wid * ROWS_PER_TILE + r
        pltpu.sync_copy(noise_hbm.at[row], noise_v)

        for p in range(3):
            shift = NBITS * p
            src = idx_a if p == 1 else idx_b
            dst = idx_b if p == 1 else idx_a

            def load_chunk(c):
                if p == 0:
                    idxv = c * L + iota
                else:
                    idxv = src[pl.ds(c * L, L)]
                kf = plsc.load_gather(noise_v, [idxv])
                kv = plsc.bitcast(kf, jnp.int32)
                d = lax.shift_right_logical(kv, shift) & (NB - 1)
                return d, idxv

            def clr(i, carry):
                hist[pl.ds(i * L, L)] = jnp.zeros((L,), jnp.int32)
                return carry

            lax.fori_loop(0, HCHUNKS, clr, jnp.int32(0))

            def histo(c, carry):
                d, idxv = load_chunk(c)
                sd, _, rank, is_end = chunk_stats(d, idxv)
                plsc.addupdate_scatter(hist, [sd], rank + 1, mask=is_end)
                return carry

            lax.fori_loop(0, CHUNKS, histo, jnp.int32(0))

            def scan(i, carry):
                v = hist[pl.ds(i * L, L)]
                incl = plsc.cumsum(v)
                hist[pl.ds(i * L, L)] = incl - v + carry
                return carry + jnp.max(incl)

            lax.fori_loop(0, HCHUNKS, scan, jnp.int32(0))

            def scat(c, carry):
                d, idxv = load_chunk(c)
                sd, sv, rank, is_end = chunk_stats(d, idxv)
                starts = plsc.load_gather(hist, [sd])
                plsc.store_scatter(dst, [starts + rank], sv)
                plsc.addupdate_scatter(hist, [sd], rank + 1, mask=is_end)
                return carry

            lax.fori_loop(0, CHUNKS, scat, jnp.int32(0))

        pltpu.sync_copy(idx_a, out_hbm.at[row])
        return jnp.int32(0)

    lax.fori_loop(0, ROWS_PER_TILE, do_row, jnp.int32(0))


def kernel(x, noise):
    del x
    perm = _argsort_rows(noise)
    return perm[:, :NUM_MASKED], perm[:, NUM_MASKED:]


# speedup vs baseline: 1.4439x; 1.0205x over previous
import functools

import jax
import jax.numpy as jnp
from jax import lax
from jax.experimental import pallas as pl
from jax.experimental.pallas import tpu as pltpu
from jax.experimental.pallas import tpu_sc as plsc

B = 128
G = 8192
NUM_MASKED = 4915
L = 16
CHUNKS = G // L
NBITS = 10
NB = 1 << NBITS
HCHUNKS = NB // L
N_WORKERS = 32
ROWS_PER_TILE = B // N_WORKERS
NWAY = 2
ROUNDS = ROWS_PER_TILE // NWAY

_mesh = plsc.VectorSubcoreMesh(core_axis_name="c", subcore_axis_name="s")

_scratch = []
for _ in range(NWAY):
    _scratch += [
        pltpu.VMEM((G,), jnp.float32),
        pltpu.VMEM((G,), jnp.int32),
        pltpu.VMEM((G,), jnp.int32),
        pltpu.VMEM((NB,), jnp.int32),
    ]


@functools.partial(
    pl.kernel,
    out_type=jax.ShapeDtypeStruct((B, G), jnp.int32),
    mesh=_mesh,
    scratch_types=_scratch,
    compiler_params=pltpu.CompilerParams(needs_layout_passes=False),
)
def _argsort_rows(noise_hbm, out_hbm, *scratch):
    noise_v = scratch[0::4]
    idx_a = scratch[1::4]
    idx_b = scratch[2::4]
    hist = scratch[3::4]

    core = lax.axis_index("c")
    sub = lax.axis_index("s")
    wid = sub * 2 + core
    iota = lax.iota(jnp.int32, L)

    cnt0, _ = plsc.scan_count(jnp.zeros((L,), jnp.int32))
    c0 = jnp.min(cnt0)

    def do_rows(r, _):
        rows = [wid * ROWS_PER_TILE + r * NWAY + q for q in range(NWAY)]
        for q in range(NWAY):
            pltpu.sync_copy(noise_hbm.at[rows[q]], noise_v[q])

        for p in range(3):
            shift = NBITS * p
            src = idx_a if p == 1 else idx_b
            dst = idx_b if p == 1 else idx_a

            def load_chunk(q, c):
                if p == 0:
                    idxv = c * L + iota
                else:
                    idxv = src[q][pl.ds(c * L, L)]
                kf = plsc.load_gather(noise_v[q], [idxv])
                kv = plsc.bitcast(kf, jnp.int32)
                d = lax.shift_right_logical(kv, shift) & (NB - 1)
                return d, idxv

            def clr(i, carry):
                for q in range(NWAY):
                    hist[q][pl.ds(i * L, L)] = jnp.zeros((L,), jnp.int32)
                return carry

            lax.fori_loop(0, HCHUNKS, clr, jnp.int32(0))

            def histo(c, carry):
                for q in range(NWAY):
                    d, _ = load_chunk(q, c)
                    cnt, last = plsc.scan_count(d)
                    plsc.addupdate_scatter(hist[q], [d], cnt - c0 + 1,
                                           mask=last)
                return carry

            lax.fori_loop(0, CHUNKS, histo, jnp.int32(0))

            def scan(i, carry):
                nxt = []
                for q in range(NWAY):
                    v = hist[q][pl.ds(i * L, L)]
                    incl = plsc.cumsum(v)
                    hist[q][pl.ds(i * L, L)] = incl - v + carry[q]
                    nxt.append(carry[q] + jnp.max(incl))
                return tuple(nxt)

            lax.fori_loop(0, HCHUNKS, scan, (jnp.int32(0),) * NWAY)

            def scat(c, carry):
                for q in range(NWAY):
                    d, idxv = load_chunk(q, c)
                    cnt, last = plsc.scan_count(d)
                    rank = cnt - c0
                    starts = plsc.load_gather(hist[q], [d])
                    plsc.store_scatter(dst[q], [starts + rank], idxv)
                    plsc.addupdate_scatter(hist[q], [d], rank + 1, mask=last)
                return carry

            lax.fori_loop(0, CHUNKS, scat, jnp.int32(0))

        for q in range(NWAY):
            pltpu.sync_copy(idx_a[q], out_hbm.at[rows[q]])
        return jnp.int32(0)

    lax.fori_loop(0, ROUNDS, do_rows, jnp.int32(0))


def kernel(x, noise):
    del x
    perm = _argsort_rows(noise)
    return perm[:, :NUM_MASKED], perm[:, NUM_MASKED:]


# speedup vs baseline: 1.4814x; 1.0259x over previous
import functools

import jax
import jax.numpy as jnp
from jax import lax
from jax.experimental import pallas as pl
from jax.experimental.pallas import tpu as pltpu
from jax.experimental.pallas import tpu_sc as plsc

B = 128
G = 8192
NUM_MASKED = 4915
L = 16
CHUNKS = G // L
NBITS = 10
NB = 1 << NBITS
HCHUNKS = NB // L
N_WORKERS = 32
ROWS_PER_TILE = B // N_WORKERS
NWAY = 4
ROUNDS = ROWS_PER_TILE // NWAY

_mesh = plsc.VectorSubcoreMesh(core_axis_name="c", subcore_axis_name="s")

_scratch = []
for _ in range(NWAY):
    _scratch += [
        pltpu.VMEM((G,), jnp.float32),
        pltpu.VMEM((G,), jnp.int32),
        pltpu.VMEM((G,), jnp.int32),
        pltpu.VMEM((NB,), jnp.int32),
    ]


@functools.partial(
    pl.kernel,
    out_type=jax.ShapeDtypeStruct((B, G), jnp.int32),
    mesh=_mesh,
    scratch_types=_scratch,
    compiler_params=pltpu.CompilerParams(needs_layout_passes=False),
)
def _argsort_rows(noise_hbm, out_hbm, *scratch):
    noise_v = scratch[0::4]
    idx_a = scratch[1::4]
    idx_b = scratch[2::4]
    hist = scratch[3::4]

    core = lax.axis_index("c")
    sub = lax.axis_index("s")
    wid = sub * 2 + core
    iota = lax.iota(jnp.int32, L)

    cnt0, _ = plsc.scan_count(jnp.zeros((L,), jnp.int32))
    c0 = jnp.min(cnt0)

    def do_rows(r, _):
        rows = [wid * ROWS_PER_TILE + r * NWAY + q for q in range(NWAY)]
        for q in range(NWAY):
            pltpu.sync_copy(noise_hbm.at[rows[q]], noise_v[q])

        for p in range(3):
            shift = NBITS * p
            src = idx_a if p == 1 else idx_b
            dst = idx_b if p == 1 else idx_a

            def load_chunk(q, c):
                if p == 0:
                    idxv = c * L + iota
                else:
                    idxv = src[q][pl.ds(c * L, L)]
                kf = plsc.load_gather(noise_v[q], [idxv])
                kv = plsc.bitcast(kf, jnp.int32)
                d = lax.shift_right_logical(kv, shift) & (NB - 1)
                return d, idxv

            def clr(i, carry):
                for q in range(NWAY):
                    hist[q][pl.ds(i * L, L)] = jnp.zeros((L,), jnp.int32)
                return carry

            lax.fori_loop(0, HCHUNKS, clr, jnp.int32(0))

            def histo(c, carry):
                for q in range(NWAY):
                    d, _ = load_chunk(q, c)
                    cnt, last = plsc.scan_count(d)
                    plsc.addupdate_scatter(hist[q], [d], cnt - c0 + 1,
                                           mask=last)
                return carry

            lax.fori_loop(0, CHUNKS, histo, jnp.int32(0))

            def scan(i, carry):
                nxt = []
                for q in range(NWAY):
                    v = hist[q][pl.ds(i * L, L)]
                    incl = plsc.cumsum(v)
                    hist[q][pl.ds(i * L, L)] = incl - v + carry[q]
                    nxt.append(carry[q] + jnp.max(incl))
                return tuple(nxt)

            lax.fori_loop(0, HCHUNKS, scan, (jnp.int32(0),) * NWAY)

            def scat(c, carry):
                for q in range(NWAY):
                    d, idxv = load_chunk(q, c)
                    cnt, last = plsc.scan_count(d)
                    rank = cnt - c0
                    starts = plsc.load_gather(hist[q], [d])
                    plsc.store_scatter(dst[q], [starts + rank], idxv)
                    plsc.addupdate_scatter(hist[q], [d], rank + 1, mask=last)
                return carry

            lax.fori_loop(0, CHUNKS, scat, jnp.int32(0))

        for q in range(NWAY):
            pltpu.sync_copy(idx_a[q], out_hbm.at[rows[q]])
        return jnp.int32(0)

    lax.fori_loop(0, ROUNDS, do_rows, jnp.int32(0))


def kernel(x, noise):
    del x
    perm = _argsort_rows(noise)
    return perm[:, :NUM_MASKED], perm[:, NUM_MASKED:]


# speedup vs baseline: 3.1514x; 2.1273x over previous
import functools

import jax
import jax.numpy as jnp
from jax import lax
from jax.experimental import pallas as pl
from jax.experimental.pallas import tpu as pltpu
from jax.experimental.pallas import tpu_sc as plsc

B = 128
G = 8192
NUM_MASKED = 4915
L = 16
CHUNKS = G // L
NBITS = 10
NB = 1 << NBITS
HCHUNKS = NB // L
N_WORKERS = 32
ROWS_PER_TILE = B // N_WORKERS
NWAY = 4
ROUNDS = ROWS_PER_TILE // NWAY

_mesh = plsc.VectorSubcoreMesh(core_axis_name="c", subcore_axis_name="s")

_scratch = []
for _ in range(NWAY):
    _scratch += [
        pltpu.VMEM((G,), jnp.float32),
        pltpu.VMEM((G,), jnp.int32),
        pltpu.VMEM((G,), jnp.int32),
        pltpu.VMEM((NB,), jnp.int32),
    ]


@functools.partial(
    pl.kernel,
    out_type=jax.ShapeDtypeStruct((B, G), jnp.int32),
    mesh=_mesh,
    scratch_types=_scratch,
    compiler_params=pltpu.CompilerParams(needs_layout_passes=False),
)
def _argsort_rows(noise_hbm, out_hbm, *scratch):
    noise_v = scratch[0::4]
    idx_a = scratch[1::4]
    idx_b = scratch[2::4]
    hist = scratch[3::4]

    core = lax.axis_index("c")
    sub = lax.axis_index("s")
    wid = sub * 2 + core
    iota = lax.iota(jnp.int32, L)

    cnt0, _ = plsc.scan_count(jnp.zeros((L,), jnp.int32))
    c0 = jnp.min(cnt0)

    def do_rows(r, _):
        rows = [wid * ROWS_PER_TILE + r * NWAY + q for q in range(NWAY)]
        for q in range(NWAY):
            pltpu.sync_copy(noise_hbm.at[rows[q]], noise_v[q])

        for p in range(3):
            shift = NBITS * p
            src = idx_a if p == 1 else idx_b
            dst = idx_b if p == 1 else idx_a

            def load_chunk(q, c):
                if p == 0:
                    idxv = c * L + iota
                else:
                    idxv = src[q][pl.ds(c * L, L)]
                kf = plsc.load_gather(noise_v[q], [idxv])
                kv = plsc.bitcast(kf, jnp.int32)
                d = lax.shift_right_logical(kv, shift) & (NB - 1)
                return d, idxv

            def clr(i, carry):
                for q in range(NWAY):
                    hist[q][pl.ds(i * L, L)] = jnp.zeros((L,), jnp.int32)
                return carry

            lax.fori_loop(0, HCHUNKS, clr, jnp.int32(0))

            ones = jnp.ones((L,), jnp.int32)

            def histo(c, carry):
                ds = [load_chunk(q, c)[0] for q in range(NWAY)]
                for q in range(NWAY):
                    plsc.addupdate_scatter(hist[q], [ds[q]], ones)
                return carry

            lax.fori_loop(0, CHUNKS, histo, jnp.int32(0))

            def scan(i, carry):
                nxt = []
                for q in range(NWAY):
                    v = hist[q][pl.ds(i * L, L)]
                    incl = plsc.cumsum(v)
                    hist[q][pl.ds(i * L, L)] = incl - v + carry[q]
                    nxt.append(carry[q] + jnp.max(incl))
                return tuple(nxt)

            lax.fori_loop(0, HCHUNKS, scan, (jnp.int32(0),) * NWAY)

            def scat(c, carry):
                loaded = [load_chunk(q, c) for q in range(NWAY)]
                cnts = [plsc.scan_count(d)[0] for d, _ in loaded]
                for q in range(NWAY):
                    d, idxv = loaded[q]
                    rank = cnts[q] - c0
                    starts = plsc.load_gather(hist[q], [d])
                    plsc.store_scatter(dst[q], [starts + rank], idxv)
                    plsc.addupdate_scatter(hist[q], [d], ones)
                return carry

            lax.fori_loop(0, CHUNKS, scat, jnp.int32(0))

        for q in range(NWAY):
            pltpu.sync_copy(idx_a[q], out_hbm.at[rows[q]])
        return jnp.int32(0)

    lax.fori_loop(0, ROUNDS, do_rows, jnp.int32(0))


def kernel(x, noise):
    del x
    perm = _argsort_rows(noise)
    return perm[:, :NUM_MASKED], perm[:, NUM_MASKED:]
